# baseline (speedup 1.0000x reference)
import functools

import jax
import jax.numpy as jnp
from jax import lax
from jax.experimental import pallas as pl
from jax.experimental.pallas import tpu as pltpu
from jax.experimental.pallas import tpu_sc as plsc

Q = 1024
D = 16
N = 100000
NT = 2048
NPAD = 100352
K = 100
KPAD = 128
NEG = -3.0e38

L = 16
NLEAF = NPAD // L
S1 = 16
NB1 = NLEAF // S1
S2 = 28
NB2 = NB1 // S2
NCH = 8
BPC = NB1 // NCH
CHW = NPAD // NCH
NWORKERS = 32
QPW = Q // NWORKERS
R0 = 13
RE = 3
RMAX = 100


def _matmul_body(q_ref, k_ref, o_ref):
    j = pl.program_id(0)
    s = lax.dot_general(
        q_ref[...], k_ref[...], (((1,), (1,)), ((), ())),
        preferred_element_type=jnp.float32)
    col = j * NT + lax.broadcasted_iota(jnp.int32, s.shape, 1)
    o_ref[...] = jnp.where(col < N, s, NEG)


def _scores(queries, keys_pad):
    qb = queries.shape[0]
    return pl.pallas_call(
        _matmul_body,
        grid=(NPAD // NT,),
        in_specs=[
            pl.BlockSpec((qb, D), lambda j: (0, 0)),
            pl.BlockSpec((NT, D), lambda j: (j, 0)),
        ],
        out_specs=pl.BlockSpec((qb, NT), lambda j: (0, j)),
        out_shape=jax.ShapeDtypeStruct((qb, NPAD), jnp.float32),
    )(queries, keys_pad)


def _store1(ref, pos, val, iota):
    blk = (pos // L) * L
    vv = ref[pl.ds(blk, L)]
    ref[pl.ds(blk, L)] = jnp.where(iota == pos - blk, val, vv)


def _tree(vals, op):
    vals = list(vals)
    while len(vals) > 1:
        nxt = [op(vals[i], vals[i + 1]) for i in range(0, len(vals) - 1, 2)]
        if len(vals) % 2:
            nxt.append(vals[-1])
        vals = nxt
    return vals[0]


def _dfs(thunks, op):
    def go(lo, hi):
        if hi - lo == 1:
            return thunks[lo]()
        mid = (lo + hi) // 2
        return op(go(lo, mid), go(mid, hi))
    return go(0, len(thunks))


def _topk_sc(scores):
    qb = scores.shape[0]
    qpw = qb // NWORKERS
    mesh = plsc.VectorSubcoreMesh(core_axis_name="c", subcore_axis_name="s")

    @functools.partial(
        pl.kernel,
        mesh=mesh,
        compiler_params=pltpu.CompilerParams(needs_layout_passes=False),
        out_type=(jax.ShapeDtypeStruct((qb, KPAD), jnp.float32),
                  jax.ShapeDtypeStruct((qb, KPAD), jnp.int32)),
        scratch_types=[
            pltpu.VMEM((NPAD,), jnp.float32),
            pltpu.VMEM((NB1 * L,), jnp.float32),
            pltpu.VMEM((NB2 * L,), jnp.float32),
            pltpu.VMEM((KPAD,), jnp.float32),
            pltpu.VMEM((KPAD,), jnp.int32),
            pltpu.VMEM(((RMAX + 4) * L,), jnp.float32),
            pltpu.VMEM(((RMAX + 4) * L,), jnp.int32),
            pltpu.SemaphoreType.DMA,
            pltpu.SemaphoreType.DMA,
            pltpu.SemaphoreType.DMA,
            pltpu.SemaphoreType.DMA,
            pltpu.SemaphoreType.DMA,
        ],
    )
    def run(scores_hbm, outv_hbm, outi_hbm, row, l1v, l2v, ov, oi,
            poolv, pooln, s0, s1, s2, s3, so):
        wid = lax.axis_index("s") * 2 + lax.axis_index("c")
        iota = lax.iota(jnp.int32, L)
        sems = (s0, s1, s2, s3)
        ndma = 2

        dnums = lax.GatherDimensionNumbers(
            offset_dims=(), collapsed_slice_dims=(0,), start_index_map=(0,))

        def shuffle(v, idx):
            return lax.gather(
                v, idx[:, None], dnums, (1,),
                mode=lax.GatherScatterMode.PROMISE_IN_BOUNDS)

        shufs = [jnp.bitwise_xor(iota, s) for s in (8, 4, 2, 1)]

        def bfly(v, op):
            for idx in shufs:
                v = op(v, shuffle(v, idx))
            return v

        def build_l1_chunk(t, _):
            base = t * (S1 * L)
            vs = [row[pl.ds(base + i * L, L)] for i in range(S1)]
            l1v[pl.ds(t * L, L)] = _tree(vs, jnp.maximum)
            return 0

        def build_l2(c, _):
            base = c * S2 * L
            vs = [l1v[pl.ds(base + t * L, L)] for t in range(S2)]
            l2v[pl.ds(c * L, L)] = _tree(vs, jnp.maximum)
            return 0

        def issue_chunk(qq, t):
            return pltpu.async_copy(
                scores_hbm.at[qq, pl.ds(t * CHW, CHW)],
                row.at[pl.ds(t * CHW, CHW)], sems[t % ndma])

        def do_query(qi, _):
            q = wid * qpw + qi
            copies = [None] * NCH
            for t in range(NCH):
                if t < ndma:
                    pltpu.make_async_copy(
                        scores_hbm.at[q, pl.ds(t * CHW, CHW)],
                        row.at[pl.ds(t * CHW, CHW)], sems[t % ndma]).wait()
                else:
                    copies[t].wait()
                if t + ndma < NCH:
                    copies[t + ndma] = issue_chunk(q, t + ndma)
                lax.fori_loop(t * BPC, (t + 1) * BPC, build_l1_chunk, 0)
            lax.fori_loop(0, NB2, build_l2, 0)

            negv = jnp.full((L,), NEG, jnp.float32)

            def comb(a, b):
                take = (b[0] > a[0]) | ((b[0] == a[0]) & (b[1] < a[1]))
                return (jnp.where(take, b[0], a[0]),
                        jnp.where(take, b[1], a[1]))

            def comb2(a, b):
                take = (b[0] > a[0]) | ((b[0] == a[0]) & (b[1] < a[1]))
                v2 = jnp.maximum(jnp.minimum(a[0], b[0]),
                                 jnp.maximum(a[2], b[2]))
                return (jnp.where(take, b[0], a[0]),
                        jnp.where(take, b[1], a[1]), v2)

            def pbfly(v, e):
                for idx in shufs:
                    vs, es = shuffle(v, idx), shuffle(e, idx)
                    take = (vs > v) | ((vs == v) & (es < e))
                    v = jnp.where(take, vs, v)
                    e = jnp.where(take, es, e)
                return v, e

            def encv(t):
                return jnp.full((L,), t, jnp.int32)

            def round_body(r, _):
                rval, rt = _dfs(
                    [lambda t=t: (l2v[pl.ds(t * L, L)], encv(t))
                     for t in range(NB2)], comb)
                base2 = rt * (S2 * L) + iota
                v1, trel, v2 = _dfs(
                    [lambda t=t: (plsc.load_gather(l1v, [base2 + t * L]),
                                  encv(t), negv) for t in range(S2)], comb2)
                bvec = rt * S2 + trel
                base3 = bvec * (S1 * L) + iota
                w1, irel, w2 = _dfs(
                    [lambda i=i: (plsc.load_gather(row, [base3 + i * L]),
                                  encv(i), negv) for i in range(S1)], comb2)
                nvec = (bvec * S1 + irel) * L + iota
                plsc.store_scatter(row, [nvec], negv)
                plsc.store_scatter(l1v, [bvec * L + iota], w2)
                plsc.store_scatter(l2v, [rt * L + iota], jnp.maximum(v2, w2))
                poolv[pl.ds(r * L, L)] = rval
                pooln[pl.ds(r * L, L)] = nvec
                poolv[pl.ds((r + 1) * L, L)] = negv
                return 0

            def do_pops():
                def pop_body(e, carry):
                    ptr, _ = carry
                    ppos = ptr * L + iota
                    heads = plsc.load_gather(poolv, [ppos])
                    hn = plsc.load_gather(pooln, [ppos])
                    hv, nsp = pbfly(heads, hn)
                    _store1(ov, e, hv, iota)
                    _store1(oi, e, nsp, iota)
                    ptr = ptr + jnp.where((heads == hv) & (hn == nsp), 1, 0)
                    return ptr, hv
                zz = jnp.zeros((L,), jnp.int32)
                _, theta = lax.fori_loop(0, K, pop_body, (zz, negv))
                return theta

            lax.fori_loop(0, R0, round_body, 0)

            def need_more(rr):
                mx = _dfs([lambda t=t: l2v[pl.ds(t * L, L)]
                           for t in range(NB2)], jnp.maximum)
                mn = bfly(mx, jnp.maximum)
                tmin = _dfs([lambda r=r: poolv[pl.ds(r * L, L)]
                             for r in range(7)], jnp.minimum)
                th = bfly(tmin, jnp.minimum)
                return (mn[0] >= th[0]) & (rr < RMAX)

            def extend(rr):
                lax.fori_loop(rr, rr + RE, round_body, 0)
                return rr + RE

            lax.while_loop(need_more, extend, jnp.int32(R0))

            @pl.when(qi + 1 < qpw)
            def _prefetch_next():
                for t in range(ndma):
                    issue_chunk(q + 1, t)

            @pl.when(qi > 0)
            def _drain_prev_out():
                pltpu.make_async_copy(ov, outv_hbm.at[q], so).wait()
                pltpu.make_async_copy(oi, outi_hbm.at[q], so).wait()

            do_pops()
            pltpu.async_copy(ov, outv_hbm.at[q], so)
            pltpu.async_copy(oi, outi_hbm.at[q], so)
            return 0

        for t in range(KPAD // L):
            ov[pl.ds(t * L, L)] = jnp.zeros((L,), jnp.float32)
            oi[pl.ds(t * L, L)] = jnp.zeros((L,), jnp.int32)
        for t in range(ndma):
            issue_chunk(wid * qpw, t)
        lax.fori_loop(0, qpw, do_query, 0)
        pltpu.make_async_copy(ov, outv_hbm.at[0], so).wait()
        pltpu.make_async_copy(oi, outi_hbm.at[0], so).wait()

    return run(scores)


SPLITS = ((0, 1024),)


def kernel(queries, keys, k):
    keys_pad = jnp.pad(keys, ((0, NPAD - N), (0, 0)))
    vparts, iparts = [], []
    parts = [_scores(queries[lo:hi], keys_pad) for lo, hi in SPLITS]
    for scores in parts:
        vals, idxs = _topk_sc(scores)
        vparts.append(vals[:, :K])
        iparts.append(idxs[:, :K])
    values = jnp.concatenate(vparts, axis=0)
    indices = jnp.concatenate(iparts, axis=0)
    indices = indices + (jnp.asarray(k, dtype=jnp.int32) - K)
    return values, indices


# speedup vs baseline: 1.0270x; 1.0270x over previous
import functools

import jax
import jax.numpy as jnp
from jax import lax
from jax.experimental import pallas as pl
from jax.experimental.pallas import tpu as pltpu
from jax.experimental.pallas import tpu_sc as plsc

Q = 1024
D = 16
N = 100000
NT = 2048
NPAD = 100352
K = 100
KPAD = 128
NEG = -3.0e38

L = 16
NLEAF = NPAD // L
S1 = 16
NB1 = NLEAF // S1
S2 = 28
NB2 = NB1 // S2
NCH = 8
BPC = NB1 // NCH
CHW = NPAD // NCH
NWORKERS = 32
QPW = Q // NWORKERS
R0 = 13
RE = 3
RMAX = 100


def _matmul_body(q_ref, k_ref, o_ref):
    j = pl.program_id(0)
    s = lax.dot_general(
        q_ref[...], k_ref[...], (((1,), (1,)), ((), ())),
        preferred_element_type=jnp.float32)
    col = j * NT + lax.broadcasted_iota(jnp.int32, s.shape, 1)
    o_ref[...] = jnp.where(col < N, s, NEG)


def _scores(queries, keys_pad):
    qb = queries.shape[0]
    return pl.pallas_call(
        _matmul_body,
        grid=(NPAD // NT,),
        in_specs=[
            pl.BlockSpec((qb, D), lambda j: (0, 0)),
            pl.BlockSpec((NT, D), lambda j: (j, 0)),
        ],
        out_specs=pl.BlockSpec((qb, NT), lambda j: (0, j)),
        out_shape=jax.ShapeDtypeStruct((qb, NPAD), jnp.float32),
    )(queries, keys_pad)


def _store1(ref, pos, val, iota):
    blk = (pos // L) * L
    vv = ref[pl.ds(blk, L)]
    ref[pl.ds(blk, L)] = jnp.where(iota == pos - blk, val, vv)


def _tree(vals, op):
    vals = list(vals)
    while len(vals) > 1:
        nxt = [op(vals[i], vals[i + 1]) for i in range(0, len(vals) - 1, 2)]
        if len(vals) % 2:
            nxt.append(vals[-1])
        vals = nxt
    return vals[0]


def _dfs(thunks, op):
    def go(lo, hi):
        if hi - lo == 1:
            return thunks[lo]()
        mid = (lo + hi) // 2
        return op(go(lo, mid), go(mid, hi))
    return go(0, len(thunks))


def _topk_sc(scores):
    qb = scores.shape[0]
    qpw = qb // NWORKERS
    mesh = plsc.VectorSubcoreMesh(core_axis_name="c", subcore_axis_name="s")

    @functools.partial(
        pl.kernel,
        mesh=mesh,
        compiler_params=pltpu.CompilerParams(needs_layout_passes=False),
        out_type=(jax.ShapeDtypeStruct((qb, KPAD), jnp.float32),
                  jax.ShapeDtypeStruct((qb, KPAD), jnp.int32)),
        scratch_types=[
            pltpu.VMEM((NPAD,), jnp.float32),
            pltpu.VMEM((NB1 * L,), jnp.float32),
            pltpu.VMEM((NB2 * L,), jnp.float32),
            pltpu.VMEM((KPAD,), jnp.float32),
            pltpu.VMEM((KPAD,), jnp.int32),
            pltpu.VMEM(((RMAX + 4) * L,), jnp.float32),
            pltpu.VMEM(((RMAX + 4) * L,), jnp.int32),
            pltpu.SemaphoreType.DMA,
            pltpu.SemaphoreType.DMA,
            pltpu.SemaphoreType.DMA,
            pltpu.SemaphoreType.DMA,
            pltpu.SemaphoreType.DMA,
        ],
    )
    def run(scores_hbm, outv_hbm, outi_hbm, row, l1v, l2v, ov, oi,
            poolv, pooln, s0, s1, s2, s3, so):
        wid = lax.axis_index("s") * 2 + lax.axis_index("c")
        iota = lax.iota(jnp.int32, L)
        sems = (s0, s1, s2, s3)
        ndma = 2

        dnums = lax.GatherDimensionNumbers(
            offset_dims=(), collapsed_slice_dims=(0,), start_index_map=(0,))

        def shuffle(v, idx):
            return lax.gather(
                v, idx[:, None], dnums, (1,),
                mode=lax.GatherScatterMode.PROMISE_IN_BOUNDS)

        shufs = [jnp.bitwise_xor(iota, s) for s in (8, 4, 2, 1)]

        def bfly(v, op):
            for idx in shufs:
                v = op(v, shuffle(v, idx))
            return v

        def build_l1_chunk(t, _):
            base = t * (S1 * L)
            vs = [row[pl.ds(base + i * L, L)] for i in range(S1)]
            l1v[pl.ds(t * L, L)] = _tree(vs, jnp.maximum)
            return 0

        def build_l2(c, _):
            base = c * S2 * L
            vs = [l1v[pl.ds(base + t * L, L)] for t in range(S2)]
            l2v[pl.ds(c * L, L)] = _tree(vs, jnp.maximum)
            return 0

        def issue_chunk(qq, t):
            return pltpu.async_copy(
                scores_hbm.at[qq, pl.ds(t * CHW, CHW)],
                row.at[pl.ds(t * CHW, CHW)], sems[t % ndma])

        def do_query(qi, _):
            q = wid * qpw + qi
            copies = [None] * NCH
            for t in range(NCH):
                if t < ndma:
                    pltpu.make_async_copy(
                        scores_hbm.at[q, pl.ds(t * CHW, CHW)],
                        row.at[pl.ds(t * CHW, CHW)], sems[t % ndma]).wait()
                else:
                    copies[t].wait()
                if t + ndma < NCH:
                    copies[t + ndma] = issue_chunk(q, t + ndma)
                lax.fori_loop(t * BPC, (t + 1) * BPC, build_l1_chunk, 0)
            lax.fori_loop(0, NB2, build_l2, 0)

            negv = jnp.full((L,), NEG, jnp.float32)

            def comb(a, b):
                take = (b[0] > a[0]) | ((b[0] == a[0]) & (b[1] < a[1]))
                return (jnp.where(take, b[0], a[0]),
                        jnp.where(take, b[1], a[1]))

            def comb2(a, b):
                take = (b[0] > a[0]) | ((b[0] == a[0]) & (b[1] < a[1]))
                v2 = jnp.maximum(jnp.minimum(a[0], b[0]),
                                 jnp.maximum(a[2], b[2]))
                return (jnp.where(take, b[0], a[0]),
                        jnp.where(take, b[1], a[1]), v2)

            def pbfly(v, e):
                for idx in shufs:
                    vs, es = shuffle(v, idx), shuffle(e, idx)
                    take = (vs > v) | ((vs == v) & (es < e))
                    v = jnp.where(take, vs, v)
                    e = jnp.where(take, es, e)
                return v, e

            def encv(t):
                return jnp.full((L,), t, jnp.int32)

            def round_body(r, _):
                rval, rt = _dfs(
                    [lambda t=t: (l2v[pl.ds(t * L, L)], encv(t))
                     for t in range(NB2)], comb)
                base2 = rt * (S2 * L) + iota
                v1, trel, v2 = _dfs(
                    [lambda t=t: (plsc.load_gather(l1v, [base2 + t * L]),
                                  encv(t), negv) for t in range(S2)], comb2)
                bvec = rt * S2 + trel
                base3 = bvec * (S1 * L) + iota
                w1, irel, w2 = _dfs(
                    [lambda i=i: (plsc.load_gather(row, [base3 + i * L]),
                                  encv(i), negv) for i in range(S1)], comb2)
                nvec = (bvec * S1 + irel) * L + iota
                plsc.store_scatter(row, [nvec], negv)
                plsc.store_scatter(l1v, [bvec * L + iota], w2)
                plsc.store_scatter(l2v, [rt * L + iota], jnp.maximum(v2, w2))
                poolv[pl.ds(r * L, L)] = rval
                pooln[pl.ds(r * L, L)] = nvec
                poolv[pl.ds((r + 1) * L, L)] = negv
                return 0

            def do_pops():
                def pop_body(e, carry):
                    ptr, _ = carry
                    ppos = ptr * L + iota
                    heads = plsc.load_gather(poolv, [ppos])
                    hn = plsc.load_gather(pooln, [ppos])
                    hv, nsp = pbfly(heads, hn)
                    _store1(ov, e, hv, iota)
                    _store1(oi, e, nsp, iota)
                    ptr = ptr + jnp.where((heads == hv) & (hn == nsp), 1, 0)
                    return ptr, hv
                zz = jnp.zeros((L,), jnp.int32)
                _, theta = lax.fori_loop(0, K, pop_body, (zz, negv))
                return theta

            lax.fori_loop(0, R0, round_body, 0)

            def need_more(rr):
                mx = _dfs([lambda t=t: l2v[pl.ds(t * L, L)]
                           for t in range(NB2)], jnp.maximum)
                mn = bfly(mx, jnp.maximum)
                tmin = _dfs([lambda r=r: poolv[pl.ds(r * L, L)]
                             for r in range(7)], jnp.minimum)
                th = bfly(tmin, jnp.minimum)
                return (mn[0] >= th[0]) & (rr < RMAX)

            def extend(rr):
                lax.fori_loop(rr, rr + RE, round_body, 0)
                return rr + RE

            lax.while_loop(need_more, extend, jnp.int32(R0))

            @pl.when(qi + 1 < qpw)
            def _prefetch_next():
                for t in range(ndma):
                    issue_chunk(q + 1, t)

            @pl.when(qi > 0)
            def _drain_prev_out():
                pltpu.make_async_copy(ov, outv_hbm.at[q], so).wait()
                pltpu.make_async_copy(oi, outi_hbm.at[q], so).wait()

            do_pops()
            pltpu.async_copy(ov, outv_hbm.at[q], so)
            pltpu.async_copy(oi, outi_hbm.at[q], so)
            return 0

        for t in range(KPAD // L):
            ov[pl.ds(t * L, L)] = jnp.zeros((L,), jnp.float32)
            oi[pl.ds(t * L, L)] = jnp.zeros((L,), jnp.int32)
        for t in range(ndma):
            issue_chunk(wid * qpw, t)
        lax.fori_loop(0, qpw, do_query, 0)
        pltpu.make_async_copy(ov, outv_hbm.at[0], so).wait()
        pltpu.make_async_copy(oi, outi_hbm.at[0], so).wait()

    return run(scores)


SPLITS = ((0, 512), (512, 1024))


def kernel(queries, keys, k):
    keys_pad = jnp.pad(keys, ((0, NPAD - N), (0, 0)))
    vparts, iparts = [], []
    parts = [_scores(queries[lo:hi], keys_pad) for lo, hi in SPLITS]
    for scores in parts:
        vals, idxs = _topk_sc(scores)
        vparts.append(vals[:, :K])
        iparts.append(idxs[:, :K])
    values = jnp.concatenate(vparts, axis=0)
    indices = jnp.concatenate(iparts, axis=0)
    indices = indices + (jnp.asarray(k, dtype=jnp.int32) - K)
    return values, indices


# speedup vs baseline: 1.0856x; 1.0570x over previous
import functools

import jax
import jax.numpy as jnp
from jax import lax
from jax.experimental import pallas as pl
from jax.experimental.pallas import tpu as pltpu
from jax.experimental.pallas import tpu_sc as plsc

Q = 1024
D = 16
N = 100000
NT = 7168
NPAD = 100352
K = 100
KPAD = 128
NEG = -3.0e38

L = 16
NLEAF = NPAD // L
S1 = 16
NB1 = NLEAF // S1
S2 = 14
NB2 = NB1 // S2
NCH = 8
BPC = NB1 // NCH
CHW = NPAD // NCH
NWORKERS = 32
QPW = Q // NWORKERS
R0 = 13
RE = 3
RMAX = 100


def _matmul_body(q_ref, k_ref, o_ref):
    j = pl.program_id(0)
    s = lax.dot_general(
        q_ref[...], k_ref[...], (((1,), (1,)), ((), ())),
        preferred_element_type=jnp.float32)
    col = j * NT + lax.broadcasted_iota(jnp.int32, s.shape, 1)
    o_ref[...] = jnp.where(col < N, s, NEG)


def _scores(queries, keys_pad):
    qb = queries.shape[0]
    return pl.pallas_call(
        _matmul_body,
        grid=(NPAD // NT,),
        in_specs=[
            pl.BlockSpec((qb, D), lambda j: (0, 0)),
            pl.BlockSpec((NT, D), lambda j: (j, 0)),
        ],
        out_specs=pl.BlockSpec((qb, NT), lambda j: (0, j)),
        out_shape=jax.ShapeDtypeStruct((qb, NPAD), jnp.float32),
    )(queries, keys_pad)


def _store1(ref, pos, val, iota):
    blk = (pos // L) * L
    vv = ref[pl.ds(blk, L)]
    ref[pl.ds(blk, L)] = jnp.where(iota == pos - blk, val, vv)


def _tree(vals, op):
    vals = list(vals)
    while len(vals) > 1:
        nxt = [op(vals[i], vals[i + 1]) for i in range(0, len(vals) - 1, 2)]
        if len(vals) % 2:
            nxt.append(vals[-1])
        vals = nxt
    return vals[0]


def _dfs(thunks, op):
    def go(lo, hi):
        if hi - lo == 1:
            return thunks[lo]()
        mid = (lo + hi) // 2
        return op(go(lo, mid), go(mid, hi))
    return go(0, len(thunks))


def _topk_sc(scores):
    qb = scores.shape[0]
    qpw = qb // NWORKERS
    mesh = plsc.VectorSubcoreMesh(core_axis_name="c", subcore_axis_name="s")

    @functools.partial(
        pl.kernel,
        mesh=mesh,
        compiler_params=pltpu.CompilerParams(needs_layout_passes=False),
        out_type=(jax.ShapeDtypeStruct((qb, KPAD), jnp.float32),
                  jax.ShapeDtypeStruct((qb, KPAD), jnp.int32)),
        scratch_types=[
            pltpu.VMEM((NPAD,), jnp.float32),
            pltpu.VMEM((NB1 * L,), jnp.float32),
            pltpu.VMEM((NB2 * L,), jnp.float32),
            pltpu.VMEM((KPAD,), jnp.float32),
            pltpu.VMEM((KPAD,), jnp.int32),
            pltpu.VMEM(((RMAX + 4) * L,), jnp.float32),
            pltpu.VMEM(((RMAX + 4) * L,), jnp.int32),
            pltpu.SemaphoreType.DMA,
            pltpu.SemaphoreType.DMA,
            pltpu.SemaphoreType.DMA,
            pltpu.SemaphoreType.DMA,
            pltpu.SemaphoreType.DMA,
        ],
    )
    def run(scores_hbm, outv_hbm, outi_hbm, row, l1v, l2v, ov, oi,
            poolv, pooln, s0, s1, s2, s3, so):
        wid = lax.axis_index("s") * 2 + lax.axis_index("c")
        iota = lax.iota(jnp.int32, L)
        sems = (s0, s1, s2, s3)
        ndma = 3

        dnums = lax.GatherDimensionNumbers(
            offset_dims=(), collapsed_slice_dims=(0,), start_index_map=(0,))

        def shuffle(v, idx):
            return lax.gather(
                v, idx[:, None], dnums, (1,),
                mode=lax.GatherScatterMode.PROMISE_IN_BOUNDS)

        shufs = [jnp.bitwise_xor(iota, s) for s in (8, 4, 2, 1)]

        def bfly(v, op):
            for idx in shufs:
                v = op(v, shuffle(v, idx))
            return v

        def build_l1_chunk(t, _):
            base = t * (S1 * L)
            vs = [row[pl.ds(base + i * L, L)] for i in range(S1)]
            l1v[pl.ds(t * L, L)] = _tree(vs, jnp.maximum)
            return 0

        def build_l2(c, _):
            base = c * S2 * L
            vs = [l1v[pl.ds(base + t * L, L)] for t in range(S2)]
            l2v[pl.ds(c * L, L)] = _tree(vs, jnp.maximum)
            return 0

        def issue_chunk(qq, t):
            return pltpu.async_copy(
                scores_hbm.at[qq, pl.ds(t * CHW, CHW)],
                row.at[pl.ds(t * CHW, CHW)], sems[t % ndma])

        def do_query(qi, _):
            q = wid * qpw + qi
            copies = [None] * NCH
            for t in range(NCH):
                if t < ndma:
                    pltpu.make_async_copy(
                        scores_hbm.at[q, pl.ds(t * CHW, CHW)],
                        row.at[pl.ds(t * CHW, CHW)], sems[t % ndma]).wait()
                else:
                    copies[t].wait()
                if t + ndma < NCH:
                    copies[t + ndma] = issue_chunk(q, t + ndma)
                lax.fori_loop(t * BPC, (t + 1) * BPC, build_l1_chunk, 0)
            lax.fori_loop(0, NB2, build_l2, 0)

            negv = jnp.full((L,), NEG, jnp.float32)

            def comb(a, b):
                take = (b[0] > a[0]) | ((b[0] == a[0]) & (b[1] < a[1]))
                return (jnp.where(take, b[0], a[0]),
                        jnp.where(take, b[1], a[1]))

            def comb2(a, b):
                take = (b[0] > a[0]) | ((b[0] == a[0]) & (b[1] < a[1]))
                v2 = jnp.maximum(jnp.minimum(a[0], b[0]),
                                 jnp.maximum(a[2], b[2]))
                return (jnp.where(take, b[0], a[0]),
                        jnp.where(take, b[1], a[1]), v2)

            def pbfly(v, e):
                for idx in shufs:
                    vs, es = shuffle(v, idx), shuffle(e, idx)
                    take = (vs > v) | ((vs == v) & (es < e))
                    v = jnp.where(take, vs, v)
                    e = jnp.where(take, es, e)
                return v, e

            def encv(t):
                return jnp.full((L,), t, jnp.int32)

            def round_body(r, _):
                rval, rt = _dfs(
                    [lambda t=t: (l2v[pl.ds(t * L, L)], encv(t))
                     for t in range(NB2)], comb)
                base2 = rt * (S2 * L) + iota
                v1, trel, v2 = _dfs(
                    [lambda t=t: (plsc.load_gather(l1v, [base2 + t * L]),
                                  encv(t), negv) for t in range(S2)], comb2)
                bvec = rt * S2 + trel
                base3 = bvec * (S1 * L) + iota
                w1, irel, w2 = _dfs(
                    [lambda i=i: (plsc.load_gather(row, [base3 + i * L]),
                                  encv(i), negv) for i in range(S1)], comb2)
                nvec = (bvec * S1 + irel) * L + iota
                plsc.store_scatter(row, [nvec], negv)
                plsc.store_scatter(l1v, [bvec * L + iota], w2)
                plsc.store_scatter(l2v, [rt * L + iota], jnp.maximum(v2, w2))
                poolv[pl.ds(r * L, L)] = rval
                pooln[pl.ds(r * L, L)] = nvec
                poolv[pl.ds((r + 1) * L, L)] = negv
                return 0

            def do_pops():
                def pop_body(e, carry):
                    ptr, _ = carry
                    ppos = ptr * L + iota
                    heads = plsc.load_gather(poolv, [ppos])
                    hn = plsc.load_gather(pooln, [ppos])
                    hv, nsp = pbfly(heads, hn)
                    _store1(ov, e, hv, iota)
                    _store1(oi, e, nsp, iota)
                    ptr = ptr + jnp.where((heads == hv) & (hn == nsp), 1, 0)
                    return ptr, hv
                zz = jnp.zeros((L,), jnp.int32)
                _, theta = lax.fori_loop(0, K, pop_body, (zz, negv))
                return theta

            lax.fori_loop(0, R0, round_body, 0)

            def need_more(rr):
                mx = _dfs([lambda t=t: l2v[pl.ds(t * L, L)]
                           for t in range(NB2)], jnp.maximum)
                mn = bfly(mx, jnp.maximum)
                tmin = _dfs([lambda r=r: poolv[pl.ds(r * L, L)]
                             for r in range(7)], jnp.minimum)
                th = bfly(tmin, jnp.minimum)
                return (mn[0] >= th[0]) & (rr < RMAX)

            def extend(rr):
                lax.fori_loop(rr, rr + RE, round_body, 0)
                return rr + RE

            lax.while_loop(need_more, extend, jnp.int32(R0))

            @pl.when(qi + 1 < qpw)
            def _prefetch_next():
                for t in range(ndma):
                    issue_chunk(q + 1, t)

            @pl.when(qi > 0)
            def _drain_prev_out():
                pltpu.make_async_copy(ov, outv_hbm.at[q], so).wait()
                pltpu.make_async_copy(oi, outi_hbm.at[q], so).wait()

            do_pops()
            pltpu.async_copy(ov, outv_hbm.at[q], so)
            pltpu.async_copy(oi, outi_hbm.at[q], so)
            return 0

        for t in range(KPAD // L):
            ov[pl.ds(t * L, L)] = jnp.zeros((L,), jnp.float32)
            oi[pl.ds(t * L, L)] = jnp.zeros((L,), jnp.int32)
        for t in range(ndma):
            issue_chunk(wid * qpw, t)
        lax.fori_loop(0, qpw, do_query, 0)
        pltpu.make_async_copy(ov, outv_hbm.at[0], so).wait()
        pltpu.make_async_copy(oi, outi_hbm.at[0], so).wait()

    return run(scores)


SPLITS = ((0, 512), (512, 1024))


def kernel(queries, keys, k):
    keys_pad = jnp.pad(keys, ((0, NPAD - N), (0, 0)))
    vparts, iparts = [], []
    parts = [_scores(queries[lo:hi], keys_pad) for lo, hi in SPLITS]
    for scores in parts:
        vals, idxs = _topk_sc(scores)
        vparts.append(vals[:, :K])
        iparts.append(idxs[:, :K])
    values = jnp.concatenate(vparts, axis=0)
    indices = jnp.concatenate(iparts, axis=0)
    indices = indices + (jnp.asarray(k, dtype=jnp.int32) - K)
    return values, indices
